# baseline (speedup 1.0000x reference)
import functools

import jax
import jax.numpy as jnp
from jax import lax
from jax.experimental import pallas as pl
from jax.experimental.pallas import tpu as pltpu
from jax.experimental.pallas import tpu_sc as plsc

_N_BEH = 3
_N_EDGES = 320000
_N_NODES = 10000
_D = 128
_CHUNK = 128
_N_TILES = 16
_EPT = _N_EDGES // _N_TILES
_NCH = 160
_PADE = _NCH * _CHUNK - _EPT
_RPT = 624
_TAIL = _N_NODES - _RPT * _N_TILES


def _sc_aggregate(emb, midx, val):
    mesh = plsc.VectorSubcoreMesh(core_axis_name="c", subcore_axis_name="s")

    @functools.partial(
        pl.kernel,
        out_type=jax.ShapeDtypeStruct((2, _N_BEH, _N_NODES, _D), jnp.float32),
        mesh=mesh,
        scratch_types=[
            pltpu.VMEM_SHARED((_N_NODES, _D), jnp.float32),
            pltpu.VMEM((2 * _CHUNK,), jnp.int32),
            pltpu.VMEM((_CHUNK,), jnp.float32),
            pltpu.VMEM((_CHUNK, _D), jnp.float32),
        ],
    )
    def agg_kernel(emb_hbm, midx_hbm, val_hbm, out_hbm,
                   acc, meta_v, va0, rows0):
        cid = lax.axis_index("c")
        sid = lax.axis_index("s")

        zeros_f = jnp.zeros((16,), jnp.float32)

        def zrow(e, _):
            for k in range(_D // 16):
                rows0[e, pl.ds(k * 16, 16)] = zeros_f
            return 0

        def scale(rr, vr):
            def grp(g, _):
                vv = vr[pl.ds(g * 16, 16)]
                for el in range(16):
                    v = jnp.full((16,), vv[el], jnp.float32)
                    e = g * 16 + el
                    for k in range(_D // 16):
                        sl = pl.ds(k * 16, 16)
                        rr[e, sl] = rr[e, sl] * v
                return 0
            lax.fori_loop(0, _CHUNK // 16, grp, 0)

        didx_ref = meta_v.at[pl.ds(0, _CHUNK)]
        sidx_ref = meta_v.at[pl.ds(_CHUNK, _CHUNK)]

        for b in range(_N_BEH):
            lax.fori_loop(0, _CHUNK, zrow, 0)
            row0 = sid * _RPT
            for z, n in ((0, _CHUNK), (1, _CHUNK), (2, _CHUNK), (3, _CHUNK),
                         (4, _RPT - 4 * _CHUNK)):
                pltpu.sync_copy(rows0.at[pl.ds(0, n)],
                                acc.at[pl.ds(row0 + z * _CHUNK, n)])

            @pl.when(sid == 0)
            def _():
                pltpu.sync_copy(rows0.at[pl.ds(0, _TAIL)],
                                acc.at[pl.ds(_N_TILES * _RPT, _TAIL)])

            plsc.subcore_barrier()

            mbase = ((cid * _N_BEH + b) * _N_TILES + sid) * _NCH * 2 * _CHUNK
            vbase = ((cid * _N_BEH + b) * _N_TILES + sid) * _NCH * _CHUNK

            def body(j, _):
                pltpu.sync_copy(
                    midx_hbm.at[pl.ds(mbase + j * 2 * _CHUNK, 2 * _CHUNK)],
                    meta_v)
                pltpu.sync_copy(
                    val_hbm.at[pl.ds(vbase + j * _CHUNK, _CHUNK)], va0)
                pltpu.sync_copy(emb_hbm.at[sidx_ref], rows0)
                scale(rows0, va0)
                pltpu.sync_copy(rows0, acc.at[didx_ref], add=True)
                return 0

            lax.fori_loop(0, _NCH, body, 0)

            plsc.subcore_barrier()

            pltpu.sync_copy(acc.at[pl.ds(row0, _RPT)],
                            out_hbm.at[cid, b, pl.ds(row0, _RPT)])

            @pl.when(sid == 0)
            def _():
                pltpu.sync_copy(
                    acc.at[pl.ds(_N_TILES * _RPT, _TAIL)],
                    out_hbm.at[cid, b, pl.ds(_N_TILES * _RPT, _TAIL)])

            plsc.subcore_barrier()

    return agg_kernel(emb, midx, val)


_ROWS_BLK = 400


def _proj_body(agg_ref, w_ref, big_ref, mean_ref):
    w = w_ref[0]
    acc = None
    for b in range(_N_BEH):
        y = jnp.dot(agg_ref[0, b], w, preferred_element_type=jnp.float32)
        big_ref[0, b] = jax.nn.sigmoid(y)
        acc = y if acc is None else acc + y
    mean_ref[0] = jax.nn.sigmoid(acc * (1.0 / _N_BEH))


def _tc_project(agg, w2):
    grid = (2, _N_NODES // _ROWS_BLK)
    return pl.pallas_call(
        _proj_body,
        grid=grid,
        in_specs=[
            pl.BlockSpec((1, _N_BEH, _ROWS_BLK, _D), lambda s, r: (s, 0, r, 0)),
            pl.BlockSpec((1, _D, _D), lambda s, r: (s, 0, 0)),
        ],
        out_specs=[
            pl.BlockSpec((1, _N_BEH, _ROWS_BLK, _D), lambda s, r: (s, 0, r, 0)),
            pl.BlockSpec((1, _ROWS_BLK, _D), lambda s, r: (s, r, 0)),
        ],
        out_shape=[
            jax.ShapeDtypeStruct((2, _N_BEH, _N_NODES, _D), jnp.float32),
            jax.ShapeDtypeStruct((2, _N_NODES, _D), jnp.float32),
        ],
    )(agg, w2)


def _pad_chunks(x):
    pad = [(0, 0)] * (x.ndim - 1) + [(0, _PADE)]
    return jnp.pad(x, pad).reshape(*x.shape[:-1], _NCH, _CHUNK)


@jax.jit
def kernel(user_embedding, item_embedding, edge_val, u_w, i_w, edge_user, edge_item):
    emb = jnp.concatenate([item_embedding, user_embedding], axis=0)
    sidx = jnp.stack([edge_item, edge_user + _N_NODES], axis=0)
    didx = jnp.stack([edge_user, edge_item], axis=0)
    val2 = jnp.broadcast_to(edge_val, (2, _N_BEH, _N_EDGES))
    s5 = _pad_chunks(sidx.reshape(2, _N_BEH, _N_TILES, _EPT))
    d5 = _pad_chunks(didx.reshape(2, _N_BEH, _N_TILES, _EPT))
    midx = jnp.stack([d5, s5], axis=4).reshape(-1)
    val2 = _pad_chunks(val2.reshape(2, _N_BEH, _N_TILES, _EPT)).reshape(-1)
    agg = _sc_aggregate(emb, midx, val2)
    w2 = jnp.stack([u_w, i_w], axis=0)
    big, mean = _tc_project(agg, w2)
    return (mean[0], mean[1], big[0], big[1])


# speedup vs baseline: 1.7638x; 1.7638x over previous
import functools

import jax
import jax.numpy as jnp
from jax import lax
from jax.experimental import pallas as pl
from jax.experimental.pallas import tpu as pltpu
from jax.experimental.pallas import tpu_sc as plsc

_N_BEH = 3
_N_EDGES = 320000
_N_NODES = 10000
_D = 128
_CHUNK = 128
_N_TILES = 16
_EPT = _N_EDGES // _N_TILES
_FULL = _EPT // _CHUNK
_REM = _EPT - _FULL * _CHUNK
_RPT = 624
_TAIL = _N_NODES - _RPT * _N_TILES


def _sc_aggregate(emb, src_idx, dst_idx, val3):
    mesh = plsc.VectorSubcoreMesh(core_axis_name="c", subcore_axis_name="s")

    @functools.partial(
        pl.kernel,
        out_type=jax.ShapeDtypeStruct((2, _N_BEH, _N_NODES, _D), jnp.float32),
        mesh=mesh,
        scratch_types=[
            pltpu.VMEM_SHARED((_N_NODES, _D), jnp.float32),
            pltpu.VMEM((_CHUNK,), jnp.int32),
            pltpu.VMEM((_CHUNK,), jnp.int32),
            pltpu.VMEM((_CHUNK,), jnp.float32),
            pltpu.VMEM((_CHUNK, _D), jnp.float32),
        ],
    )
    def agg_kernel(emb_hbm, sidx_hbm, didx_hbm, val_hbm, out_hbm,
                   acc, sidx_v, didx_v, val_v, rows_v):
        cid = lax.axis_index("c")
        sid = lax.axis_index("s")

        zeros_f = jnp.zeros((16,), jnp.float32)

        def scale_grp(g, _):
            vv = val_v[pl.ds(g * 16, 16)]
            for el in range(16):
                v = jnp.full((16,), vv[el], jnp.float32)
                e = g * 16 + el
                for k in range(_D // 16):
                    sl = pl.ds(k * 16, 16)
                    rows_v[e, sl] = rows_v[e, sl] * v
            return 0

        def zrow(e, _):
            for k in range(_D // 16):
                rows_v[e, pl.ds(k * 16, 16)] = zeros_f
            return 0

        def do_chunk(ebase, off, nzero):
            pltpu.sync_copy(sidx_hbm.at[pl.ds(ebase + off, _CHUNK)], sidx_v)
            pltpu.sync_copy(didx_hbm.at[pl.ds(ebase + off, _CHUNK)], didx_v)
            vbase = (ebase % (_N_BEH * _N_EDGES))
            pltpu.sync_copy(val_hbm.at[pl.ds(vbase + off, _CHUNK)], val_v)
            for k in range(nzero // 16):
                val_v[pl.ds(k * 16, 16)] = zeros_f
            pltpu.sync_copy(emb_hbm.at[sidx_v], rows_v)
            lax.fori_loop(0, _CHUNK // 16, scale_grp, 0)
            pltpu.sync_copy(rows_v, acc.at[didx_v], add=True)

        for b in range(_N_BEH):
            lax.fori_loop(0, _CHUNK, zrow, 0)
            row0 = sid * _RPT
            for z, n in ((0, _CHUNK), (1, _CHUNK), (2, _CHUNK), (3, _CHUNK),
                         (4, _RPT - 4 * _CHUNK)):
                pltpu.sync_copy(rows_v.at[pl.ds(0, n)],
                                acc.at[pl.ds(row0 + z * _CHUNK, n)])

            @pl.when(sid == 0)
            def _():
                pltpu.sync_copy(rows_v.at[pl.ds(0, _TAIL)],
                                acc.at[pl.ds(_N_TILES * _RPT, _TAIL)])

            plsc.subcore_barrier()

            ebase = (cid * _N_BEH + b) * _N_EDGES + sid * _EPT

            def chunk_body(j, _):
                do_chunk(ebase, j * _CHUNK, 0)
                return 0

            lax.fori_loop(0, _FULL, chunk_body, 0)
            do_chunk(ebase, _EPT - _CHUNK, _CHUNK - _REM)

            plsc.subcore_barrier()

            pltpu.sync_copy(acc.at[pl.ds(row0, _RPT)],
                            out_hbm.at[cid, b, pl.ds(row0, _RPT)])

            @pl.when(sid == 0)
            def _():
                pltpu.sync_copy(
                    acc.at[pl.ds(_N_TILES * _RPT, _TAIL)],
                    out_hbm.at[cid, b, pl.ds(_N_TILES * _RPT, _TAIL)])

            plsc.subcore_barrier()

    return agg_kernel(emb, src_idx, dst_idx, val3)


_ROWS_BLK = 400


def _proj_body(agg_ref, w_ref, big_ref, mean_ref):
    w = w_ref[0]
    acc = None
    for b in range(_N_BEH):
        y = jnp.dot(agg_ref[0, b], w, preferred_element_type=jnp.float32)
        big_ref[0, b] = jax.nn.sigmoid(y)
        acc = y if acc is None else acc + y
    mean_ref[0] = jax.nn.sigmoid(acc * (1.0 / _N_BEH))


def _tc_project(agg, w2):
    grid = (2, _N_NODES // _ROWS_BLK)
    return pl.pallas_call(
        _proj_body,
        grid=grid,
        in_specs=[
            pl.BlockSpec((1, _N_BEH, _ROWS_BLK, _D), lambda s, r: (s, 0, r, 0)),
            pl.BlockSpec((1, _D, _D), lambda s, r: (s, 0, 0)),
        ],
        out_specs=[
            pl.BlockSpec((1, _N_BEH, _ROWS_BLK, _D), lambda s, r: (s, 0, r, 0)),
            pl.BlockSpec((1, _ROWS_BLK, _D), lambda s, r: (s, r, 0)),
        ],
        out_shape=[
            jax.ShapeDtypeStruct((2, _N_BEH, _N_NODES, _D), jnp.float32),
            jax.ShapeDtypeStruct((2, _N_NODES, _D), jnp.float32),
        ],
    )(agg, w2)


@jax.jit
def kernel(user_embedding, item_embedding, edge_val, u_w, i_w, edge_user, edge_item):
    emb = jnp.concatenate([item_embedding, user_embedding], axis=0)
    src_idx = jnp.stack([edge_item, edge_user + _N_NODES], axis=0).reshape(-1)
    dst_idx = jnp.stack([edge_user, edge_item], axis=0).reshape(-1)
    agg = _sc_aggregate(emb, src_idx, dst_idx, edge_val.reshape(-1))
    w2 = jnp.stack([u_w, i_w], axis=0)
    big, mean = _tc_project(agg, w2)
    return (mean[0], mean[1], big[0], big[1])
